# baseline (speedup 1.0000x reference)
import functools

import jax
import jax.numpy as jnp
from jax import lax
from jax.experimental import pallas as pl
from jax.experimental.pallas import tpu as pltpu
from jax.experimental.pallas import tpu_sc as plsc

N = 1_000_000
D = 16
NLAB = 32
NC = 2
NS = 16
NW = NC * NS
C = 2048
NFULL = N // C
MID = 512
MID_OFF = NFULL * C
TAIL = 64
TAIL_OFF = N - TAIL
GROUPS = C // 16
MAXT = (NFULL + NW - 1) // NW


def _sc_body(feat_hbm, pos_hbm, label_hbm, ftail_hbm,
             out_hbm, otail_hbm,
             fb0, pb0, pb1, pb2, labelbuf, tailbuf,
             sp0, sp1, sp2, soc):
    wid = lax.axis_index("s") * NC + lax.axis_index("c")

    lanes = jnp.arange(16, dtype=jnp.int32)
    col0 = jnp.zeros((16,), jnp.int32)
    col1 = jnp.ones((16,), jnp.int32)
    col2 = jnp.full((16,), 2, jnp.int32)
    ones = jnp.ones((16,), jnp.float32)
    fvec = jnp.zeros((16,), jnp.bool_)

    pbs = (pb0, pb1, pb2)
    sps = (sp0, sp1, sp2)

    pltpu.sync_copy(label_hbm, labelbuf)
    blk = []
    for half in range(2):
        rows16 = half * 16 + lanes
        l0 = plsc.load_gather(labelbuf, [rows16, col0])
        l1 = plsc.load_gather(labelbuf, [rows16, col1])
        l2 = plsc.load_gather(labelbuf, [rows16, col2])
        lk = l0 * 65536 + l1 * 256 + l2
        blk.extend(lk[j] for j in range(16))

    def group_match(pb, g):
        base16 = g * 16
        p0 = pb[pl.ds(base16, 16)]
        p1 = pb[pl.ds(C + base16, 16)]
        p2 = pb[pl.ds(2 * C + base16, 16)]
        key = p0 * 65536 + p1 * 256 + p2
        m = key == blk[0]
        for j in range(1, NLAB):
            m = m | (key == blk[j])
        return m

    def scan_chunk(pb, ngroups):
        def body(g, acc):
            return acc | group_match(pb, g)
        return lax.fori_loop(0, ngroups, body, fvec)

    def fix_chunk(fb, pb, ngroups):
        def body(g, c2):
            m = group_match(pb, g)

            @pl.when(jnp.any(m))
            def _():
                cols = g * 16 + lanes
                for r in range(D):
                    plsc.store_scatter(
                        fb, [jnp.full((16,), r, jnp.int32), cols],
                        ones, mask=m)

            return c2
        lax.fori_loop(0, ngroups, body, 0)

    def issue_pos(start, slot):
        pb, sem = pbs[slot], sps[slot]
        pltpu.async_copy(pos_hbm.at[pl.ds(start, C)], pb.at[pl.ds(0, C)], sem)
        pltpu.async_copy(pos_hbm.at[pl.ds(N + start, C)],
                         pb.at[pl.ds(C, C)], sem)
        pltpu.async_copy(pos_hbm.at[pl.ds(2 * N + start, C)],
                         pb.at[pl.ds(2 * C, C)], sem)

    def wait_pos(start, slot):
        pb, sem = pbs[slot], sps[slot]
        pltpu.make_async_copy(pos_hbm.at[pl.ds(start, C)],
                              pb.at[pl.ds(0, C)], sem).wait()
        pltpu.make_async_copy(pos_hbm.at[pl.ds(N + start, C)],
                              pb.at[pl.ds(C, C)], sem).wait()
        pltpu.make_async_copy(pos_hbm.at[pl.ds(2 * N + start, C)],
                              pb.at[pl.ds(2 * C, C)], sem).wait()

    def cid(t):
        return (t * NW + wid) * C

    nt = jnp.where(wid < NFULL % NW, NFULL // NW + 1, NFULL // NW)

    issue_pos(cid(0), 0)
    issue_pos(cid(1), 1)

    def q_body(q, cnt):
        for slot in range(3):
            t = 3 * q + slot
            valid = t < nt
            start = cid(jnp.minimum(t, nt - 1))

            @pl.when(valid)
            def _():
                wait_pos(start, slot)

            has = jnp.any(scan_chunk(pbs[slot], GROUPS))

            @pl.when(valid & jnp.logical_not(has))
            def _():
                pltpu.async_copy(feat_hbm.at[:, pl.ds(start, C)],
                                 out_hbm.at[:, pl.ds(start, C)], soc)

            @pl.when(valid & has)
            def _():
                pltpu.sync_copy(feat_hbm.at[:, pl.ds(start, C)], fb0)
                fix_chunk(fb0, pbs[slot], GROUPS)
                pltpu.sync_copy(fb0, out_hbm.at[:, pl.ds(start, C)])

            cnt = cnt + jnp.where(valid & jnp.logical_not(has), 1, 0)

            @pl.when(t + 2 < nt)
            def _():
                issue_pos(cid(t + 2), (slot + 2) % 3)

        return cnt

    cnt = lax.fori_loop(0, (MAXT + 2) // 3, q_body, jnp.int32(0))

    def drain(i, c):
        pltpu.make_async_copy(feat_hbm.at[:, pl.ds(0, C)],
                              out_hbm.at[:, pl.ds(0, C)], soc).wait()
        return c

    lax.fori_loop(0, cnt, drain, 0)

    @pl.when(wid == NFULL % NW)
    def _():
        pltpu.sync_copy(pos_hbm.at[pl.ds(MID_OFF, MID)], pb0.at[pl.ds(0, MID)])
        pltpu.sync_copy(pos_hbm.at[pl.ds(N + MID_OFF, MID)],
                        pb0.at[pl.ds(C, MID)])
        pltpu.sync_copy(pos_hbm.at[pl.ds(2 * N + MID_OFF, MID)],
                        pb0.at[pl.ds(2 * C, MID)])
        pltpu.sync_copy(feat_hbm.at[:, pl.ds(MID_OFF, MID)],
                        fb0.at[:, pl.ds(0, MID)])
        fix_chunk(fb0, pb0, MID // 16)
        pltpu.sync_copy(fb0.at[:, pl.ds(0, MID)],
                        out_hbm.at[:, pl.ds(MID_OFF, MID)])

    @pl.when(wid == NFULL % NW + 1)
    def _():
        pltpu.sync_copy(pos_hbm.at[pl.ds(TAIL_OFF, TAIL)],
                        pb0.at[pl.ds(0, TAIL)])
        pltpu.sync_copy(pos_hbm.at[pl.ds(N + TAIL_OFF, TAIL)],
                        pb0.at[pl.ds(C, TAIL)])
        pltpu.sync_copy(pos_hbm.at[pl.ds(2 * N + TAIL_OFF, TAIL)],
                        pb0.at[pl.ds(2 * C, TAIL)])
        pltpu.sync_copy(ftail_hbm, tailbuf)
        fix_chunk(tailbuf, pb0, TAIL // 16)
        pltpu.sync_copy(tailbuf, otail_hbm)


def kernel(features, positions, label):
    ft = features.T
    pflat = positions.astype(jnp.int32).T.reshape(3 * N)
    label = label.astype(jnp.int32)
    ftail = lax.slice(features, (TAIL_OFF, 0), (N, D)).T
    mesh = plsc.VectorSubcoreMesh(core_axis_name="c", subcore_axis_name="s")
    f = functools.partial(
        pl.kernel,
        mesh=mesh,
        out_type=(jax.ShapeDtypeStruct((D, N), jnp.float32),
                  jax.ShapeDtypeStruct((D, TAIL), jnp.float32)),
        scratch_types=[
            pltpu.VMEM((D, C), jnp.float32),
            pltpu.VMEM((3 * C,), jnp.int32),
            pltpu.VMEM((3 * C,), jnp.int32),
            pltpu.VMEM((3 * C,), jnp.int32),
            pltpu.VMEM((NLAB, 3), jnp.int32),
            pltpu.VMEM((D, TAIL), jnp.float32),
            pltpu.SemaphoreType.DMA,
            pltpu.SemaphoreType.DMA,
            pltpu.SemaphoreType.DMA,
            pltpu.SemaphoreType.DMA,
        ],
        compiler_params=pltpu.CompilerParams(needs_layout_passes=False),
    )(_sc_body)
    out, otail = f(ft, pflat, label, ftail)
    return lax.dynamic_update_slice(out.T, otail.T, (TAIL_OFF, 0))


# speedup vs baseline: 18.3465x; 18.3465x over previous
import functools

import jax
import jax.numpy as jnp
from jax import lax
from jax.experimental import pallas as pl
from jax.experimental.pallas import tpu as pltpu
from jax.experimental.pallas import tpu_sc as plsc

N = 1_000_000
D = 16
NLAB = 32
NC = 2
NS = 16
NW = NC * NS
C = 1536
NFULL = N // C
TAIL = 64
TAIL_OFF = N - TAIL
GROUPS = C // 16
MAXT = (NFULL + NW - 1) // NW
NSLOT = 4


def _sc_body(feat_hbm, pos_hbm, label_hbm, ftail_hbm,
             out_hbm, otail_hbm,
             fb0, fb1, fb2, fb3, pb0, pb1, pb2, pb3, labelbuf, tailbuf,
             si0, si1, si2, si3, so0, so1, so2, so3):
    wid = lax.axis_index("s") * NC + lax.axis_index("c")

    lanes = jnp.arange(16, dtype=jnp.int32)
    col0 = jnp.zeros((16,), jnp.int32)
    col1 = jnp.ones((16,), jnp.int32)
    col2 = jnp.full((16,), 2, jnp.int32)
    ones = jnp.ones((16,), jnp.float32)
    fvec = jnp.zeros((16,), jnp.bool_)

    fbs = (fb0, fb1, fb2, fb3)
    pbs = (pb0, pb1, pb2, pb3)
    sis = (si0, si1, si2, si3)
    sos = (so0, so1, so2, so3)

    pltpu.sync_copy(label_hbm, labelbuf)
    blk = []
    for half in range(2):
        rows16 = half * 16 + lanes
        l0 = plsc.load_gather(labelbuf, [rows16, col0])
        l1 = plsc.load_gather(labelbuf, [rows16, col1])
        l2 = plsc.load_gather(labelbuf, [rows16, col2])
        lk = l0 * 65536 + l1 * 256 + l2
        blk.extend(lk[j] for j in range(16))

    def group_match(pb, g):
        base16 = g * 16
        p0 = pb[pl.ds(base16, 16)]
        p1 = pb[pl.ds(C + base16, 16)]
        p2 = pb[pl.ds(2 * C + base16, 16)]
        key = p0 * 65536 + p1 * 256 + p2
        m = key == blk[0]
        for j in range(1, NLAB):
            m = m | (key == blk[j])
        return m

    def scan_chunk(pb, ngroups):
        def body(g, acc):
            return acc | group_match(pb, g)
        return lax.fori_loop(0, ngroups, body, fvec)

    def fix_chunk(fb, pb, ngroups):
        def body(g, c2):
            m = group_match(pb, g)

            @pl.when(jnp.any(m))
            def _():
                cols = g * 16 + lanes
                for r in range(D):
                    plsc.store_scatter(
                        fb, [jnp.full((16,), r, jnp.int32), cols],
                        ones, mask=m)

            return c2
        lax.fori_loop(0, ngroups, body, 0)

    def issue_in(start, slot):
        fb, pb, sem = fbs[slot], pbs[slot], sis[slot]
        pltpu.async_copy(pos_hbm.at[pl.ds(start, C)], pb.at[pl.ds(0, C)], sem)
        pltpu.async_copy(pos_hbm.at[pl.ds(N + start, C)],
                         pb.at[pl.ds(C, C)], sem)
        pltpu.async_copy(pos_hbm.at[pl.ds(2 * N + start, C)],
                         pb.at[pl.ds(2 * C, C)], sem)
        pltpu.async_copy(feat_hbm.at[:, pl.ds(start, C)], fb, sem)

    def wait_in(start, slot):
        fb, pb, sem = fbs[slot], pbs[slot], sis[slot]
        pltpu.make_async_copy(pos_hbm.at[pl.ds(start, C)],
                              pb.at[pl.ds(0, C)], sem).wait()
        pltpu.make_async_copy(pos_hbm.at[pl.ds(N + start, C)],
                              pb.at[pl.ds(C, C)], sem).wait()
        pltpu.make_async_copy(pos_hbm.at[pl.ds(2 * N + start, C)],
                              pb.at[pl.ds(2 * C, C)], sem).wait()
        pltpu.make_async_copy(feat_hbm.at[:, pl.ds(start, C)], fb, sem).wait()

    def issue_out(start, slot):
        pltpu.async_copy(fbs[slot], out_hbm.at[:, pl.ds(start, C)], sos[slot])

    def wait_out(slot):
        pltpu.make_async_copy(fbs[slot], out_hbm.at[:, pl.ds(0, C)],
                              sos[slot]).wait()

    def compute(slot):
        fb, pb = fbs[slot], pbs[slot]
        acc = scan_chunk(pb, GROUPS)

        @pl.when(jnp.any(acc))
        def _():
            fix_chunk(fb, pb, GROUPS)

    def cid(t):
        return (t * NW + wid) * C

    nt = jnp.where(wid < NFULL % NW, NFULL // NW + 1, NFULL // NW)

    issue_in(cid(0), 0)
    issue_in(cid(1), 1)
    issue_in(cid(2), 2)

    def q_body(q, carry):
        for slot in range(NSLOT):
            t = NSLOT * q + slot

            @pl.when(t < nt)
            def _():
                wait_in(cid(t), slot)
                compute(slot)
                issue_out(cid(t), slot)

                @pl.when(t + 3 < nt)
                def _():
                    nslot = (slot + 3) % NSLOT

                    @pl.when(t >= 1)
                    def _():
                        wait_out(nslot)

                    issue_in(cid(t + 3), nslot)

        return carry

    lax.fori_loop(0, (MAXT + NSLOT - 1) // NSLOT, q_body, 0)

    wait_out(0)
    wait_out(1)
    wait_out(2)
    wait_out(3)

    @pl.when(wid == NFULL % NW)
    def _():
        pltpu.sync_copy(pos_hbm.at[pl.ds(TAIL_OFF, TAIL)],
                        pb0.at[pl.ds(0, TAIL)])
        pltpu.sync_copy(pos_hbm.at[pl.ds(N + TAIL_OFF, TAIL)],
                        pb0.at[pl.ds(C, TAIL)])
        pltpu.sync_copy(pos_hbm.at[pl.ds(2 * N + TAIL_OFF, TAIL)],
                        pb0.at[pl.ds(2 * C, TAIL)])
        pltpu.sync_copy(ftail_hbm, tailbuf)
        fix_chunk(tailbuf, pb0, TAIL // 16)
        pltpu.sync_copy(tailbuf, otail_hbm)


def kernel(features, positions, label):
    ft = features.T
    pflat = positions.astype(jnp.int32).T.reshape(3 * N)
    label = label.astype(jnp.int32)
    ftail = lax.slice(features, (TAIL_OFF, 0), (N, D)).T
    mesh = plsc.VectorSubcoreMesh(core_axis_name="c", subcore_axis_name="s")
    f = functools.partial(
        pl.kernel,
        mesh=mesh,
        out_type=(jax.ShapeDtypeStruct((D, N), jnp.float32),
                  jax.ShapeDtypeStruct((D, TAIL), jnp.float32)),
        scratch_types=[
            pltpu.VMEM((D, C), jnp.float32),
            pltpu.VMEM((D, C), jnp.float32),
            pltpu.VMEM((D, C), jnp.float32),
            pltpu.VMEM((D, C), jnp.float32),
            pltpu.VMEM((3 * C,), jnp.int32),
            pltpu.VMEM((3 * C,), jnp.int32),
            pltpu.VMEM((3 * C,), jnp.int32),
            pltpu.VMEM((3 * C,), jnp.int32),
            pltpu.VMEM((NLAB, 3), jnp.int32),
            pltpu.VMEM((D, TAIL), jnp.float32),
            pltpu.SemaphoreType.DMA,
            pltpu.SemaphoreType.DMA,
            pltpu.SemaphoreType.DMA,
            pltpu.SemaphoreType.DMA,
            pltpu.SemaphoreType.DMA,
            pltpu.SemaphoreType.DMA,
            pltpu.SemaphoreType.DMA,
            pltpu.SemaphoreType.DMA,
        ],
        compiler_params=pltpu.CompilerParams(needs_layout_passes=False),
    )(_sc_body)
    out, otail = f(ft, pflat, label, ftail)
    return lax.dynamic_update_slice(out.T, otail.T, (TAIL_OFF, 0))
